# baseline (speedup 1.0000x reference)
import functools

import jax
import jax.numpy as jnp
from jax import lax
from jax.experimental import pallas as pl
from jax.experimental.pallas import tpu as pltpu
from jax.experimental.pallas import tpu_sc as plsc

_K = 1024
_D = 64
_B = 16
_H = 32
_W = 32
_BETA = 0.25
_HW = _H * _W
_N = _B * _HW

_NW = 32
_RPW = _N // _NW


def _vq_body(z_ref, cb_ref, idx_ref, loss_ref):
    zb = z_ref[0]
    cb = cb_ref[...]
    z2 = jnp.sum(zb * zb, axis=0, keepdims=True)
    c2 = jnp.sum(cb * cb, axis=1, keepdims=True)
    s2 = jax.lax.dot_general(
        cb, zb + zb, (((1,), (0,)), ((), ())),
        preferred_element_type=jnp.float32)
    d = (z2 + c2) - s2
    dmin = jnp.min(d, axis=0, keepdims=True)
    kio = jax.lax.broadcasted_iota(jnp.int32, d.shape, 0).astype(jnp.float32)
    idxf = jnp.min(jnp.where(d == dmin, kio, float(_K)), axis=0, keepdims=True)
    i = pl.program_id(0)
    idx_ref[pl.ds(i, 1), :] = idxf.astype(jnp.int32)
    part = jnp.sum(dmin).reshape(1, 1)

    @pl.when(i == 0)
    def _init():
        loss_ref[...] = jnp.zeros((1, 1), jnp.float32)

    loss_ref[...] += part

    @pl.when(i == _B - 1)
    def _finish():
        loss_ref[...] = loss_ref[...] * ((1.0 + _BETA) / float(_N * _D))


def _gather_body(table_hbm, idx_hbm, out_hbm, idx_v, rows_v, sem):
    wid = lax.axis_index("s") * 2 + lax.axis_index("c")
    base = wid * _RPW
    pltpu.sync_copy(idx_hbm.at[pl.ds(base, _RPW)], idx_v)
    pltpu.async_copy(table_hbm.at[idx_v], rows_v, sem).wait()
    pltpu.sync_copy(rows_v, out_hbm.at[pl.ds(base, _RPW)])


def kernel(z, codebook):
    Bz, Dz, Hz, Wz = z.shape
    z3 = z.reshape(Bz, Dz, Hz * Wz)
    indices, loss11 = pl.pallas_call(
        _vq_body,
        grid=(_B,),
        in_specs=[
            pl.BlockSpec((1, _D, _HW), lambda i: (i, 0, 0)),
            pl.BlockSpec((_K, _D), lambda i: (0, 0)),
        ],
        out_specs=[
            pl.BlockSpec((_B, _HW), lambda i: (0, 0)),
            pl.BlockSpec((1, 1), lambda i: (0, 0)),
        ],
        out_shape=[
            jax.ShapeDtypeStruct((_B, _HW), jnp.int32),
            jax.ShapeDtypeStruct((1, 1), jnp.float32),
        ],
    )(z3, codebook)

    sc_gather = functools.partial(
        pl.kernel,
        mesh=plsc.VectorSubcoreMesh(core_axis_name="c", subcore_axis_name="s"),
        out_type=jax.ShapeDtypeStruct((_N, 128), jnp.float32),
        scratch_types=[
            pltpu.VMEM((_RPW,), jnp.int32),
            pltpu.VMEM((_RPW, 128), jnp.float32),
            pltpu.SemaphoreType.DMA,
        ],
    )(_gather_body)
    cb_pad = jnp.pad(codebook, ((0, 0), (0, 128 - _D)))
    zq_rows = sc_gather(cb_pad, indices.reshape(_N))

    z_q_st = jnp.transpose(
        zq_rows[:, :_D].reshape(Bz, Hz, Wz, Dz), (0, 3, 1, 2))
    return (z_q_st, loss11[0, 0], indices)


# speedup vs baseline: 1.3507x; 1.3507x over previous
import jax
import jax.numpy as jnp
from jax.experimental import pallas as pl

_K = 1024
_D = 64
_B = 16
_H = 32
_W = 32
_BETA = 0.25
_HW = _H * _W
_N = _B * _HW
_BPS = 2


def _vq_body(z_ref, cb_ref, zq_ref, idx_ref, loss_ref):
    cb = cb_ref[...]
    c2 = jnp.sum(cb * cb, axis=1, keepdims=True)
    i = pl.program_id(0)
    part = jnp.zeros((1, 1), jnp.float32)
    for j in range(_BPS):
        zb = z_ref[j]
        z2 = jnp.sum(zb * zb, axis=0, keepdims=True)
        s2 = jax.lax.dot_general(
            cb, zb + zb, (((1,), (0,)), ((), ())),
            preferred_element_type=jnp.float32)
        d = (z2 + c2) - s2
        dmin = jnp.min(d, axis=0, keepdims=True)
        kio = jax.lax.broadcasted_iota(jnp.int32, d.shape, 0).astype(jnp.float32)
        idxf = jnp.min(jnp.where(d == dmin, kio, float(_K)), axis=0, keepdims=True)
        oh = (kio == idxf).astype(jnp.float32)
        zq = jax.lax.dot_general(
            cb, oh, (((0,), (0,)), ((), ())),
            preferred_element_type=jnp.float32)
        zq_ref[j] = zq
        idx_ref[pl.ds(i * _BPS + j, 1), :] = idxf.astype(jnp.int32)
        part = part + jnp.sum((zq - zb) ** 2).reshape(1, 1)

    @pl.when(i == 0)
    def _init():
        loss_ref[...] = jnp.zeros((1, 1), jnp.float32)

    loss_ref[...] += part

    @pl.when(i == _B // _BPS - 1)
    def _finish():
        loss_ref[...] = loss_ref[...] * ((1.0 + _BETA) / float(_N * _D))


def kernel(z, codebook):
    Bz, Dz, Hz, Wz = z.shape
    z3 = z.reshape(Bz, Dz, Hz * Wz)
    zq3, indices, loss11 = pl.pallas_call(
        _vq_body,
        grid=(_B // _BPS,),
        in_specs=[
            pl.BlockSpec((_BPS, _D, _HW), lambda i: (i, 0, 0)),
            pl.BlockSpec((_K, _D), lambda i: (0, 0)),
        ],
        out_specs=[
            pl.BlockSpec((_BPS, _D, _HW), lambda i: (i, 0, 0)),
            pl.BlockSpec((_B, _HW), lambda i: (0, 0)),
            pl.BlockSpec((1, 1), lambda i: (0, 0)),
        ],
        out_shape=[
            jax.ShapeDtypeStruct((_B, _D, _HW), jnp.float32),
            jax.ShapeDtypeStruct((_B, _HW), jnp.int32),
            jax.ShapeDtypeStruct((1, 1), jnp.float32),
        ],
    )(z3, codebook)
    z_q_st = zq3.reshape(Bz, Dz, Hz, Wz)
    return (z_q_st, loss11[0, 0], indices)
